# baseline (speedup 1.0000x reference)
import jax
import jax.numpy as jnp
from jax import lax
from jax.experimental import pallas as pl
from jax.experimental.pallas import tpu as pltpu
from jax.experimental.pallas import tpu_sc as plsc

ROWS = 128
N = 32768
LANES = 16
NV = N // LANES
NC = 2
NS = 16
ROWS_PER_WORKER = ROWS // (NC * NS)
RADIX = 256
VB = 16
BLOCKS = NV // VB
RV = RADIX // LANES


def _to_monotone(x):
  m = jnp.int32(-2147483648)
  s = lax.shift_right_arithmetic(x, 31)
  return lax.bitwise_xor(x, lax.bitwise_or(s, m))


def _from_monotone(u):
  m = jnp.int32(-2147483648)
  s = lax.shift_right_arithmetic(u, 31)
  return lax.bitwise_xor(u, lax.bitwise_or(lax.bitwise_not(s), m))


def _zero16():
  return jnp.zeros((LANES,), jnp.int32)


def _digit(u, shift):
  d = u if shift == 0 else lax.shift_right_logical(u, shift)
  return lax.bitwise_and(d, jnp.int32(RADIX - 1))


def _sort_body(in_hbm, out_hbm, k0, k1, blk, tot):
  wid = lax.axis_index("s") * NC + lax.axis_index("c")

  def zero_blk(i):
    blk[pl.ds(i * LANES, LANES)] = _zero16()

  def make_hist(src, shift, mapped):
    def hist(i):
      sl = pl.ds(i * LANES, LANES)
      u = src[sl]
      if not mapped:
        u = _to_monotone(u)
        src[sl] = u
      d = _digit(u, shift)
      c, last = plsc.scan_count(d)
      base = lax.shift_left(lax.shift_right_logical(i, 4), 8)
      plsc.addupdate_scatter(blk, [d + base], c, mask=last)
      plsc.addupdate_scatter(tot, [d], c, mask=last)
    return hist

  def excl_scan(j, carry):
    sl = pl.ds(j * LANES, LANES)
    t = tot[sl]
    inc = plsc.cumsum(t)
    tot[sl] = inc - t + carry
    return carry + jnp.sum(t)

  def rel_sweep(b, run):
    new_run = []
    for j in range(RV):
      sl = pl.ds(b * RADIX + j * LANES, LANES)
      t = blk[sl]
      blk[sl] = run[j]
      new_run.append(run[j] + t)
    return tuple(new_run)

  def make_perm(src, dst, shift, finalize):
    def perm(b):
      cbase = b * RADIX
      for v in range(VB):
        sl = pl.ds((b * VB + v) * LANES, LANES)
        u = src[sl]
        d = _digit(u, shift)
        cnt, last = plsc.scan_count(d)
        base = plsc.load_gather(blk, [d + cbase])
        pos = base + cnt - 1
        out = _from_monotone(u) if finalize else u
        plsc.store_scatter(dst, [pos], out)
        plsc.addupdate_scatter(blk, [d + cbase], cnt, mask=last)
      for j in range(RV):
        blk[pl.ds(b * RADIX + j * LANES, LANES)] = _zero16()
    return perm

  def do_pass(src, dst, shift, mapped, finalize):
    plsc.parallel_loop(0, NV, step=1, unroll=4)(make_hist(src, shift, mapped))
    lax.fori_loop(0, RV, excl_scan, jnp.int32(0))
    seed = []
    for j in range(RV):
      sl = pl.ds(j * LANES, LANES)
      seed.append(tot[sl])
      tot[sl] = _zero16()
    lax.fori_loop(0, BLOCKS, rel_sweep, tuple(seed))
    plsc.parallel_loop(0, BLOCKS, step=1)(make_perm(src, dst, shift, finalize))

  def row_body(r, c):
    row = wid * ROWS_PER_WORKER + r
    pltpu.sync_copy(in_hbm.at[row], k0)
    do_pass(k0, k1, 0, False, False)
    do_pass(k1, k0, 8, True, False)
    do_pass(k0, k1, 16, True, False)
    do_pass(k1, k0, 24, True, True)
    pltpu.sync_copy(k0, out_hbm.at[row])
    return c

  plsc.parallel_loop(0, NV, step=1, unroll=4)(zero_blk)
  for j in range(RV):
    tot[pl.ds(j * LANES, LANES)] = _zero16()
  lax.fori_loop(0, ROWS_PER_WORKER, row_body, jnp.int32(0))


@jax.jit
def kernel(inputs):
  xi = lax.bitcast_convert_type(inputs, jnp.int32)
  mesh = plsc.VectorSubcoreMesh(
      core_axis_name="c", subcore_axis_name="s", num_cores=NC,
      num_subcores=NS)
  sorted_i = pl.kernel(
      _sort_body,
      out_type=jax.ShapeDtypeStruct((ROWS, N), jnp.int32),
      mesh=mesh,
      scratch_types=[
          pltpu.VMEM((N,), jnp.int32),
          pltpu.VMEM((N,), jnp.int32),
          pltpu.VMEM((BLOCKS * RADIX,), jnp.int32),
          pltpu.VMEM((RADIX,), jnp.int32),
      ],
      compiler_params=pltpu.CompilerParams(needs_layout_passes=False),
  )(xi)
  return lax.bitcast_convert_type(sorted_i, jnp.float32)
